# Initial kernel scaffold; baseline (speedup 1.0000x reference)
import jax
import jax.numpy as jnp
from jax.experimental import pallas as pl


def kernel(logits, text_emb, image_emb, logit_bias):
    raise NotImplementedError("write your pallas kernel here")


# speedup vs baseline: 37.2816x; 37.2816x over previous
import functools

import jax
import jax.numpy as jnp
from jax import lax
from jax.experimental import pallas as pl
from jax.experimental.pallas import tpu as pltpu
from jax.experimental.pallas import tpu_sc as plsc

B = 4096
D = 64
ALPHA = 0.1
NEG_RANK = (B * B - B - 1) // 2

BLK = 256
NBLK = B // BLK

NW = 32
L = 16
NPER = B * B // NW
CHUNK = 2048
NCH = NPER // CHUNK

INF_KEY = 0xFF800000


def _monotone_key(x):
    u = lax.bitcast_convert_type(x, jnp.uint32)
    return jnp.where((u >> 31) == 0, u | jnp.uint32(0x80000000), ~u)


def _log_sigmoid(z):
    return jnp.minimum(z, 0.0) - jnp.log(1.0 + jnp.exp(-jnp.abs(z)))


def _prep_body(logits_ref, text_ref, img_ref, bias_ref,
               keys_ref, selsim_ref, base_ref):
    i = pl.program_id(0)
    l = logits_ref[...]
    t = text_ref[...]
    im = img_ref[...]
    bias = bias_ref[0, 0]

    raw = lax.dot_general(t, im, (((1,), (1,)), ((), ())),
                          preferred_element_type=jnp.float32)

    rows = i * BLK + lax.broadcasted_iota(jnp.int32, (BLK, B), 0)
    cols = lax.broadcasted_iota(jnp.int32, (BLK, B), 1)
    diag = rows == cols

    lb = l + bias
    z = jnp.where(diag, lb, -lb)

    @pl.when(i == 0)
    def _():
        base_ref[0, 0] = 0.0

    base_ref[0, 0] += jnp.sum(_log_sigmoid(z))

    rm = jnp.max(jnp.where(diag, -jnp.inf, raw), axis=1)
    selsim_ref[0, 0, :] = rm

    ratio = l / (raw + 1e-8)
    key = _monotone_key(ratio)
    key = jnp.where(diag, jnp.uint32(INF_KEY), key)
    keys_ref[...] = key.reshape(BLK * B // 128, 128)


def _prep(logits, text_emb, image_emb, bias2d):
    return pl.pallas_call(
        _prep_body,
        grid=(NBLK,),
        in_specs=[
            pl.BlockSpec((BLK, B), lambda i: (i, 0)),
            pl.BlockSpec((BLK, D), lambda i: (i, 0)),
            pl.BlockSpec((B, D), lambda i: (0, 0)),
            pl.BlockSpec(memory_space=pltpu.SMEM),
        ],
        out_specs=[
            pl.BlockSpec((BLK * B // 128, 128), lambda i: (i, 0)),
            pl.BlockSpec((1, 1, BLK), lambda i: (i, 0, 0)),
            pl.BlockSpec(memory_space=pltpu.SMEM),
        ],
        out_shape=[
            jax.ShapeDtypeStruct((B * B // 128, 128), jnp.uint32),
            jax.ShapeDtypeStruct((NBLK, 1, BLK), jnp.float32),
            jax.ShapeDtypeStruct((1, 1), jnp.float32),
        ],
    )(logits, text_emb, image_emb, bias2d)


def _make_sc_pass(shift, nbins, pshift):
    mesh = plsc.VectorSubcoreMesh(core_axis_name="c", subcore_axis_name="s",
                                  num_cores=2, num_subcores=16)

    def body(keys_hbm, pref_hbm, hist_hbm, buf0, buf1, hist, prefv,
             sem0, sem1):
        cid = lax.axis_index("c")
        sid = lax.axis_index("s")
        wid = sid * 2 + cid
        base = wid * NPER

        zeros = jnp.zeros((L,), jnp.int32)

        @pl.loop(0, L * nbins, step=L, unroll=8)
        def _(z_i):
            hist[pl.ds(z_i, L)] = zeros

        lane_base = lax.iota(jnp.int32, L) * nbins
        ones = jnp.ones((L,), jnp.int32)

        if pshift is not None:
            pltpu.sync_copy(pref_hbm, prefv)
            pvu = plsc.bitcast(prefv[...], jnp.uint32)

        def process(buf):
            @pl.loop(0, CHUNK, step=L, unroll=8)
            def _(j):
                k = buf[pl.ds(j, L)]
                bin_ = ((k >> shift) & jnp.uint32(nbins - 1)).astype(jnp.int32)
                addr = lane_base + bin_
                if pshift is None:
                    plsc.addupdate_scatter(hist, [addr], ones)
                else:
                    plsc.addupdate_scatter(hist, [addr], ones,
                                           mask=(k >> pshift) == pvu)

        pltpu.make_async_copy(
            keys_hbm.at[pl.ds(base, CHUNK)], buf0, sem0).start()
        pltpu.make_async_copy(
            keys_hbm.at[pl.ds(base + CHUNK, CHUNK)], buf1, sem1).start()

        @pl.loop(0, NCH, step=2)
        def _(c):
            pltpu.make_async_copy(
                keys_hbm.at[pl.ds(base, CHUNK)], buf0, sem0).wait()
            process(buf0)

            @pl.when(c + 2 < NCH)
            def _():
                off = pl.multiple_of(base + (c + 2) * CHUNK, CHUNK)
                pltpu.make_async_copy(
                    keys_hbm.at[pl.ds(off, CHUNK)], buf0, sem0).start()

            pltpu.make_async_copy(
                keys_hbm.at[pl.ds(base, CHUNK)], buf1, sem1).wait()
            process(buf1)

            @pl.when(c + 3 < NCH)
            def _():
                off = pl.multiple_of(base + (c + 3) * CHUNK, CHUNK)
                pltpu.make_async_copy(
                    keys_hbm.at[pl.ds(off, CHUNK)], buf1, sem1).start()

        pltpu.sync_copy(hist, hist_hbm.at[wid])

    return pl.kernel(
        body,
        out_type=jax.ShapeDtypeStruct((NW, L * nbins), jnp.int32),
        mesh=mesh,
        scratch_types=[
            pltpu.VMEM((CHUNK,), jnp.uint32),
            pltpu.VMEM((CHUNK,), jnp.uint32),
            pltpu.VMEM((L * nbins,), jnp.int32),
            pltpu.VMEM((L,), jnp.int32),
            pltpu.SemaphoreType.DMA,
            pltpu.SemaphoreType.DMA,
        ],
        compiler_params=pltpu.CompilerParams(needs_layout_passes=False),
    )


_sc_pass1 = _make_sc_pass(20, 4096, None)
_sc_pass2 = _make_sc_pass(8, 4096, 20)
_sc_pass3 = _make_sc_pass(0, 256, 8)


def _cumsum_1d(c, n):
    cum = c
    k = 1
    while k < n:
        shifted = jnp.concatenate(
            [jnp.zeros((1, k), jnp.int32), cum[:, :n - k]], axis=1)
        cum = cum + shifted
        k *= 2
    return cum


def _select_body(nbins, hist_ref, rank_ref, pref_ref,
                 pref16_ref, rank_out, pref_out):
    h = hist_ref[...]
    counts = jnp.sum(jnp.sum(h, axis=0), axis=0)
    cum = _cumsum_1d(counts.reshape(1, nbins), nbins)
    r = rank_ref[0, 0]
    le = cum <= r
    b = jnp.sum(le.astype(jnp.int32))
    prev = jnp.max(jnp.where(le, cum, 0))
    newpref = pref_ref[0, 0] * nbins + b
    rank_out[0, 0] = r - prev
    pref_out[0, 0] = newpref
    pref16_ref[...] = jnp.full((L,), newpref, jnp.int32)


def _make_select(nbins):
    return pl.pallas_call(
        functools.partial(_select_body, nbins),
        in_specs=[
            pl.BlockSpec((NW, L, nbins), lambda: (0, 0, 0)),
            pl.BlockSpec(memory_space=pltpu.SMEM),
            pl.BlockSpec(memory_space=pltpu.SMEM),
        ],
        out_specs=[
            pl.BlockSpec((L,), lambda: (0,)),
            pl.BlockSpec(memory_space=pltpu.SMEM),
            pl.BlockSpec(memory_space=pltpu.SMEM),
        ],
        out_shape=[
            jax.ShapeDtypeStruct((L,), jnp.int32),
            jax.ShapeDtypeStruct((1, 1), jnp.int32),
            jax.ShapeDtypeStruct((1, 1), jnp.int32),
        ],
    )


_select1 = _make_select(4096)
_select2 = _make_select(4096)


def _final_body(hist_ref, rank_ref, pref_ref, base_ref, selsim_ref, out_ref):
    h = hist_ref[...]
    counts = jnp.sum(jnp.sum(h, axis=0), axis=0)
    cum = _cumsum_1d(counts.reshape(1, 256), 256)
    r = rank_ref[0, 0]
    b = jnp.sum((cum <= r).astype(jnp.int32))
    key = pref_ref[0, 0] * 256 + b
    u = jnp.where(key >= 0, ~key, key & jnp.int32(0x7FFFFFFF))
    scale = lax.bitcast_convert_type(u, jnp.float32)

    s = selsim_ref[...]
    select_loss = -jnp.sum(_log_sigmoid(-scale * s)) / B
    base_loss = -base_ref[0, 0] / (B * B)
    out_ref[0, 0] = base_loss + ALPHA * select_loss


_final = pl.pallas_call(
    _final_body,
    in_specs=[
        pl.BlockSpec((NW, L, 256), lambda: (0, 0, 0)),
        pl.BlockSpec(memory_space=pltpu.SMEM),
        pl.BlockSpec(memory_space=pltpu.SMEM),
        pl.BlockSpec(memory_space=pltpu.SMEM),
        pl.BlockSpec((NBLK, 1, BLK), lambda: (0, 0, 0)),
    ],
    out_specs=pl.BlockSpec(memory_space=pltpu.SMEM),
    out_shape=jax.ShapeDtypeStruct((1, 1), jnp.float32),
)


def kernel(logits, text_emb, image_emb, logit_bias):
    bias2d = jnp.reshape(logit_bias, (1, 1)).astype(jnp.float32)
    keys2d, selsim, base_sum = _prep(logits, text_emb, image_emb, bias2d)
    keys = keys2d.reshape(B * B)

    rank0 = jnp.full((1, 1), NEG_RANK, jnp.int32)
    pref0 = jnp.zeros((1, 1), jnp.int32)
    dummy16 = jnp.zeros((L,), jnp.int32)

    hist1 = _sc_pass1(keys, dummy16)
    pref16, rank1, pref1 = _select1(hist1.reshape(NW, L, 4096), rank0, pref0)
    hist2 = _sc_pass2(keys, pref16)
    pref16b, rank2, pref2 = _select2(hist2.reshape(NW, L, 4096), rank1, pref1)
    hist3 = _sc_pass3(keys, pref16b)
    total = _final(hist3.reshape(NW, L, 256), rank2, pref2, base_sum, selsim)
    return jnp.reshape(total, ())


# speedup vs baseline: 109.7350x; 2.9434x over previous
import functools

import jax
import jax.numpy as jnp
from jax import lax
from jax.experimental import pallas as pl
from jax.experimental.pallas import tpu as pltpu
from jax.experimental.pallas import tpu_sc as plsc

B = 4096
D = 64
ALPHA = 0.1
NEG_RANK = (B * B - B - 1) // 2

BLK = 256
NBLK = B // BLK

NW = 32
L = 16
NPER = B * B // NW
CHUNK = 8192
NCH = NPER // CHUNK

INF_KEY = 0xFF800000


def _monotone_key(x):
    u = lax.bitcast_convert_type(x, jnp.uint32)
    return jnp.where((u >> 31) == 0, u | jnp.uint32(0x80000000), ~u)


def _log_sigmoid(z):
    return jnp.minimum(z, 0.0) - jnp.log(1.0 + jnp.exp(-jnp.abs(z)))


def _prep_body(logits_ref, text_ref, img_ref, bias_ref,
               keys_ref, selsim_ref, base_ref):
    i = pl.program_id(0)
    l = logits_ref[...]
    t = text_ref[...]
    im = img_ref[...]
    bias = bias_ref[0, 0]

    raw = lax.dot_general(t, im, (((1,), (1,)), ((), ())),
                          preferred_element_type=jnp.float32)

    rows = i * BLK + lax.broadcasted_iota(jnp.int32, (BLK, B), 0)
    cols = lax.broadcasted_iota(jnp.int32, (BLK, B), 1)
    diag = rows == cols

    lb = l + bias
    z = jnp.where(diag, lb, -lb)

    @pl.when(i == 0)
    def _():
        base_ref[0, 0] = 0.0

    base_ref[0, 0] += jnp.sum(_log_sigmoid(z))

    rm = jnp.max(jnp.where(diag, -jnp.inf, raw), axis=1)
    selsim_ref[0, 0, :] = rm

    ratio = l / (raw + 1e-8)
    key = _monotone_key(ratio)
    key = jnp.where(diag, jnp.uint32(INF_KEY), key)
    keys_ref[...] = key.reshape(BLK * B // 128, 128)


def _prep(logits, text_emb, image_emb, bias2d):
    return pl.pallas_call(
        _prep_body,
        grid=(NBLK,),
        in_specs=[
            pl.BlockSpec((BLK, B), lambda i: (i, 0)),
            pl.BlockSpec((BLK, D), lambda i: (i, 0)),
            pl.BlockSpec((B, D), lambda i: (0, 0)),
            pl.BlockSpec(memory_space=pltpu.SMEM),
        ],
        out_specs=[
            pl.BlockSpec((BLK * B // 128, 128), lambda i: (i, 0)),
            pl.BlockSpec((1, 1, BLK), lambda i: (i, 0, 0)),
            pl.BlockSpec(memory_space=pltpu.SMEM),
        ],
        out_shape=[
            jax.ShapeDtypeStruct((B * B // 128, 128), jnp.uint32),
            jax.ShapeDtypeStruct((NBLK, 1, BLK), jnp.float32),
            jax.ShapeDtypeStruct((1, 1), jnp.float32),
        ],
    )(logits, text_emb, image_emb, bias2d)


def _make_sc_pass(shift, nbins, pshift):
    mesh = plsc.VectorSubcoreMesh(core_axis_name="c", subcore_axis_name="s",
                                  num_cores=2, num_subcores=16)

    def body(keys_hbm, pref_hbm, hist_hbm, buf0, buf1, hist, prefv,
             sem0, sem1):
        cid = lax.axis_index("c")
        sid = lax.axis_index("s")
        wid = sid * 2 + cid
        base = wid * NPER

        zeros = jnp.zeros((L,), jnp.int32)

        @plsc.parallel_loop(0, L * nbins, step=L, unroll=8)
        def _(z_i):
            hist[pl.ds(z_i, L)] = zeros

        lane_base = lax.iota(jnp.int32, L) * nbins
        ones = jnp.ones((L,), jnp.int32)

        if pshift is not None:
            pltpu.sync_copy(pref_hbm, prefv)
            pvu = plsc.bitcast(prefv[...], jnp.uint32)

        def process(buf):
            @plsc.parallel_loop(0, CHUNK, step=L, unroll=8)
            def _(j):
                k = buf[pl.ds(j, L)]
                bin_ = ((k >> shift) & jnp.uint32(nbins - 1)).astype(jnp.int32)
                addr = lane_base + bin_
                if pshift is None:
                    plsc.addupdate_scatter(hist, [addr], ones)
                else:
                    plsc.addupdate_scatter(hist, [addr], ones,
                                           mask=(k >> pshift) == pvu)

        pltpu.make_async_copy(
            keys_hbm.at[pl.ds(base, CHUNK)], buf0, sem0).start()
        pltpu.make_async_copy(
            keys_hbm.at[pl.ds(base + CHUNK, CHUNK)], buf1, sem1).start()

        @pl.loop(0, NCH, step=2)
        def _(c):
            pltpu.make_async_copy(
                keys_hbm.at[pl.ds(base, CHUNK)], buf0, sem0).wait()
            process(buf0)

            @pl.when(c + 2 < NCH)
            def _():
                off = pl.multiple_of(base + (c + 2) * CHUNK, CHUNK)
                pltpu.make_async_copy(
                    keys_hbm.at[pl.ds(off, CHUNK)], buf0, sem0).start()

            pltpu.make_async_copy(
                keys_hbm.at[pl.ds(base, CHUNK)], buf1, sem1).wait()
            process(buf1)

            @pl.when(c + 3 < NCH)
            def _():
                off = pl.multiple_of(base + (c + 3) * CHUNK, CHUNK)
                pltpu.make_async_copy(
                    keys_hbm.at[pl.ds(off, CHUNK)], buf1, sem1).start()

        pltpu.sync_copy(hist, hist_hbm.at[wid])

    return pl.kernel(
        body,
        out_type=jax.ShapeDtypeStruct((NW, L * nbins), jnp.int32),
        mesh=mesh,
        scratch_types=[
            pltpu.VMEM((CHUNK,), jnp.uint32),
            pltpu.VMEM((CHUNK,), jnp.uint32),
            pltpu.VMEM((L * nbins,), jnp.int32),
            pltpu.VMEM((L,), jnp.int32),
            pltpu.SemaphoreType.DMA,
            pltpu.SemaphoreType.DMA,
        ],
        compiler_params=pltpu.CompilerParams(needs_layout_passes=False),
    )


_sc_pass1 = _make_sc_pass(20, 4096, None)
_sc_pass2 = _make_sc_pass(8, 4096, 20)
_sc_pass3 = _make_sc_pass(0, 256, 8)


def _cumsum_1d(c, n):
    cum = c
    k = 1
    while k < n:
        shifted = jnp.concatenate(
            [jnp.zeros((1, k), jnp.int32), cum[:, :n - k]], axis=1)
        cum = cum + shifted
        k *= 2
    return cum


def _select_body(nbins, hist_ref, rank_ref, pref_ref,
                 pref16_ref, rank_out, pref_out):
    h = hist_ref[...]
    counts = jnp.sum(jnp.sum(h, axis=0), axis=0)
    cum = _cumsum_1d(counts.reshape(1, nbins), nbins)
    r = rank_ref[0, 0]
    le = cum <= r
    b = jnp.sum(le.astype(jnp.int32))
    prev = jnp.max(jnp.where(le, cum, 0))
    newpref = pref_ref[0, 0] * nbins + b
    rank_out[0, 0] = r - prev
    pref_out[0, 0] = newpref
    pref16_ref[...] = jnp.full((L,), newpref, jnp.int32)


def _make_select(nbins):
    return pl.pallas_call(
        functools.partial(_select_body, nbins),
        in_specs=[
            pl.BlockSpec((NW, L, nbins), lambda: (0, 0, 0)),
            pl.BlockSpec(memory_space=pltpu.SMEM),
            pl.BlockSpec(memory_space=pltpu.SMEM),
        ],
        out_specs=[
            pl.BlockSpec((L,), lambda: (0,)),
            pl.BlockSpec(memory_space=pltpu.SMEM),
            pl.BlockSpec(memory_space=pltpu.SMEM),
        ],
        out_shape=[
            jax.ShapeDtypeStruct((L,), jnp.int32),
            jax.ShapeDtypeStruct((1, 1), jnp.int32),
            jax.ShapeDtypeStruct((1, 1), jnp.int32),
        ],
    )


_select1 = _make_select(4096)
_select2 = _make_select(4096)


def _final_body(hist_ref, rank_ref, pref_ref, base_ref, selsim_ref, out_ref):
    h = hist_ref[...]
    counts = jnp.sum(jnp.sum(h, axis=0), axis=0)
    cum = _cumsum_1d(counts.reshape(1, 256), 256)
    r = rank_ref[0, 0]
    b = jnp.sum((cum <= r).astype(jnp.int32))
    key = pref_ref[0, 0] * 256 + b
    u = jnp.where(key >= 0, ~key, key & jnp.int32(0x7FFFFFFF))
    scale = lax.bitcast_convert_type(u, jnp.float32)

    s = selsim_ref[...]
    select_loss = -jnp.sum(_log_sigmoid(-scale * s)) / B
    base_loss = -base_ref[0, 0] / (B * B)
    out_ref[0, 0] = base_loss + ALPHA * select_loss


_final = pl.pallas_call(
    _final_body,
    in_specs=[
        pl.BlockSpec((NW, L, 256), lambda: (0, 0, 0)),
        pl.BlockSpec(memory_space=pltpu.SMEM),
        pl.BlockSpec(memory_space=pltpu.SMEM),
        pl.BlockSpec(memory_space=pltpu.SMEM),
        pl.BlockSpec((NBLK, 1, BLK), lambda: (0, 0, 0)),
    ],
    out_specs=pl.BlockSpec(memory_space=pltpu.SMEM),
    out_shape=jax.ShapeDtypeStruct((1, 1), jnp.float32),
)


def kernel(logits, text_emb, image_emb, logit_bias):
    bias2d = jnp.reshape(logit_bias, (1, 1)).astype(jnp.float32)
    keys2d, selsim, base_sum = _prep(logits, text_emb, image_emb, bias2d)
    keys = keys2d.reshape(B * B)

    rank0 = jnp.full((1, 1), NEG_RANK, jnp.int32)
    pref0 = jnp.zeros((1, 1), jnp.int32)
    dummy16 = jnp.zeros((L,), jnp.int32)

    hist1 = _sc_pass1(keys, dummy16)
    pref16, rank1, pref1 = _select1(hist1.reshape(NW, L, 4096), rank0, pref0)
    hist2 = _sc_pass2(keys, pref16)
    pref16b, rank2, pref2 = _select2(hist2.reshape(NW, L, 4096), rank1, pref1)
    hist3 = _sc_pass3(keys, pref16b)
    total = _final(hist3.reshape(NW, L, 256), rank2, pref2, base_sum, selsim)
    return jnp.reshape(total, ())


# speedup vs baseline: 120.0412x; 1.0939x over previous
import functools

import jax
import jax.numpy as jnp
from jax import lax
from jax.experimental import pallas as pl
from jax.experimental.pallas import tpu as pltpu
from jax.experimental.pallas import tpu_sc as plsc

B = 4096
D = 64
ALPHA = 0.1
NEG_RANK = (B * B - B - 1) // 2

BLK = 256
NBLK = B // BLK

NW = 32
L = 16
NPER = B * B // NW
CHUNK = 16384
NCH = NPER // CHUNK

INF_KEY = 0xFF800000


def _monotone_key(x):
    u = lax.bitcast_convert_type(x, jnp.uint32)
    return jnp.where((u >> 31) == 0, u | jnp.uint32(0x80000000), ~u)


def _log_sigmoid(z):
    return jnp.minimum(z, 0.0) - jnp.log(1.0 + jnp.exp(-jnp.abs(z)))


def _prep_body(logits_ref, text_ref, img_ref, bias_ref,
               keys_ref, selsim_ref, base_ref):
    i = pl.program_id(0)
    l = logits_ref[...]
    t = text_ref[...]
    im = img_ref[...]
    bias = bias_ref[0, 0]

    raw = lax.dot_general(t, im, (((1,), (1,)), ((), ())),
                          preferred_element_type=jnp.float32)

    rows = i * BLK + lax.broadcasted_iota(jnp.int32, (BLK, B), 0)
    cols = lax.broadcasted_iota(jnp.int32, (BLK, B), 1)
    diag = rows == cols

    lb = l + bias
    z = jnp.where(diag, lb, -lb)

    @pl.when(i == 0)
    def _():
        base_ref[0, 0] = 0.0

    base_ref[0, 0] += jnp.sum(_log_sigmoid(z))

    rm = jnp.max(jnp.where(diag, -jnp.inf, raw), axis=1)
    selsim_ref[0, 0, :] = rm

    ratio = l / (raw + 1e-8)
    key = _monotone_key(ratio)
    key = jnp.where(diag, jnp.uint32(INF_KEY), key)
    keys_ref[...] = key.reshape(BLK * B // 128, 128)


def _prep(logits, text_emb, image_emb, bias2d):
    return pl.pallas_call(
        _prep_body,
        grid=(NBLK,),
        in_specs=[
            pl.BlockSpec((BLK, B), lambda i: (i, 0)),
            pl.BlockSpec((BLK, D), lambda i: (i, 0)),
            pl.BlockSpec((B, D), lambda i: (0, 0)),
            pl.BlockSpec(memory_space=pltpu.SMEM),
        ],
        out_specs=[
            pl.BlockSpec((BLK * B // 128, 128), lambda i: (i, 0)),
            pl.BlockSpec((1, 1, BLK), lambda i: (i, 0, 0)),
            pl.BlockSpec(memory_space=pltpu.SMEM),
        ],
        out_shape=[
            jax.ShapeDtypeStruct((B * B // 128, 128), jnp.uint32),
            jax.ShapeDtypeStruct((NBLK, 1, BLK), jnp.float32),
            jax.ShapeDtypeStruct((1, 1), jnp.float32),
        ],
    )(logits, text_emb, image_emb, bias2d)


def _make_sc_pass(shift, nbins, pshift):
    mesh = plsc.VectorSubcoreMesh(core_axis_name="c", subcore_axis_name="s",
                                  num_cores=2, num_subcores=16)

    def body(keys_hbm, pref_hbm, hist_hbm, buf0, buf1, hist, prefv,
             sem0, sem1):
        cid = lax.axis_index("c")
        sid = lax.axis_index("s")
        wid = sid * 2 + cid
        base = wid * NPER

        zeros = jnp.zeros((L,), jnp.int32)

        @plsc.parallel_loop(0, L * nbins, step=L, unroll=8)
        def _(z_i):
            hist[pl.ds(z_i, L)] = zeros

        lane_base = lax.iota(jnp.int32, L) * nbins
        ones = jnp.ones((L,), jnp.int32)

        if pshift is not None:
            pltpu.sync_copy(pref_hbm, prefv)
            pvu = plsc.bitcast(prefv[...], jnp.uint32)

        def process(buf):
            @plsc.parallel_loop(0, CHUNK, step=L, unroll=8)
            def _(j):
                k = buf[pl.ds(j, L)]
                bin_ = ((k >> shift) & jnp.uint32(nbins - 1)).astype(jnp.int32)
                addr = lane_base + bin_
                if pshift is None:
                    plsc.addupdate_scatter(hist, [addr], ones)
                else:
                    plsc.addupdate_scatter(hist, [addr], ones,
                                           mask=(k >> pshift) == pvu)

        pltpu.make_async_copy(
            keys_hbm.at[pl.ds(base, CHUNK)], buf0, sem0).start()
        pltpu.make_async_copy(
            keys_hbm.at[pl.ds(base + CHUNK, CHUNK)], buf1, sem1).start()

        @pl.loop(0, NCH, step=2)
        def _(c):
            pltpu.make_async_copy(
                keys_hbm.at[pl.ds(base, CHUNK)], buf0, sem0).wait()
            process(buf0)

            @pl.when(c + 2 < NCH)
            def _():
                off = pl.multiple_of(base + (c + 2) * CHUNK, CHUNK)
                pltpu.make_async_copy(
                    keys_hbm.at[pl.ds(off, CHUNK)], buf0, sem0).start()

            pltpu.make_async_copy(
                keys_hbm.at[pl.ds(base, CHUNK)], buf1, sem1).wait()
            process(buf1)

            @pl.when(c + 3 < NCH)
            def _():
                off = pl.multiple_of(base + (c + 3) * CHUNK, CHUNK)
                pltpu.make_async_copy(
                    keys_hbm.at[pl.ds(off, CHUNK)], buf1, sem1).start()

        pltpu.sync_copy(hist, hist_hbm.at[wid])

    return pl.kernel(
        body,
        out_type=jax.ShapeDtypeStruct((NW, L * nbins), jnp.int32),
        mesh=mesh,
        scratch_types=[
            pltpu.VMEM((CHUNK,), jnp.uint32),
            pltpu.VMEM((CHUNK,), jnp.uint32),
            pltpu.VMEM((L * nbins,), jnp.int32),
            pltpu.VMEM((L,), jnp.int32),
            pltpu.SemaphoreType.DMA,
            pltpu.SemaphoreType.DMA,
        ],
        compiler_params=pltpu.CompilerParams(needs_layout_passes=False),
    )


_sc_pass1 = _make_sc_pass(20, 4096, None)
_sc_pass2 = _make_sc_pass(8, 4096, 20)
_sc_pass3 = _make_sc_pass(0, 256, 8)


def _cumsum_1d(c, n):
    cum = c
    k = 1
    while k < n:
        shifted = jnp.concatenate(
            [jnp.zeros((1, k), jnp.int32), cum[:, :n - k]], axis=1)
        cum = cum + shifted
        k *= 2
    return cum


def _select_body(nbins, hist_ref, rank_ref, pref_ref,
                 pref16_ref, rank_out, pref_out):
    h = hist_ref[...]
    counts = jnp.sum(jnp.sum(h, axis=0), axis=0)
    cum = _cumsum_1d(counts.reshape(1, nbins), nbins)
    r = rank_ref[0, 0]
    le = cum <= r
    b = jnp.sum(le.astype(jnp.int32))
    prev = jnp.max(jnp.where(le, cum, 0))
    newpref = pref_ref[0, 0] * nbins + b
    rank_out[0, 0] = r - prev
    pref_out[0, 0] = newpref
    pref16_ref[...] = jnp.full((L,), newpref, jnp.int32)


def _make_select(nbins):
    return pl.pallas_call(
        functools.partial(_select_body, nbins),
        in_specs=[
            pl.BlockSpec((NW, L, nbins), lambda: (0, 0, 0)),
            pl.BlockSpec(memory_space=pltpu.SMEM),
            pl.BlockSpec(memory_space=pltpu.SMEM),
        ],
        out_specs=[
            pl.BlockSpec((L,), lambda: (0,)),
            pl.BlockSpec(memory_space=pltpu.SMEM),
            pl.BlockSpec(memory_space=pltpu.SMEM),
        ],
        out_shape=[
            jax.ShapeDtypeStruct((L,), jnp.int32),
            jax.ShapeDtypeStruct((1, 1), jnp.int32),
            jax.ShapeDtypeStruct((1, 1), jnp.int32),
        ],
    )


_select1 = _make_select(4096)
_select2 = _make_select(4096)


def _final_body(hist_ref, rank_ref, pref_ref, base_ref, selsim_ref, out_ref):
    h = hist_ref[...]
    counts = jnp.sum(jnp.sum(h, axis=0), axis=0)
    cum = _cumsum_1d(counts.reshape(1, 256), 256)
    r = rank_ref[0, 0]
    b = jnp.sum((cum <= r).astype(jnp.int32))
    key = pref_ref[0, 0] * 256 + b
    u = jnp.where(key >= 0, ~key, key & jnp.int32(0x7FFFFFFF))
    scale = lax.bitcast_convert_type(u, jnp.float32)

    s = selsim_ref[...]
    select_loss = -jnp.sum(_log_sigmoid(-scale * s)) / B
    base_loss = -base_ref[0, 0] / (B * B)
    out_ref[0, 0] = base_loss + ALPHA * select_loss


_final = pl.pallas_call(
    _final_body,
    in_specs=[
        pl.BlockSpec((NW, L, 256), lambda: (0, 0, 0)),
        pl.BlockSpec(memory_space=pltpu.SMEM),
        pl.BlockSpec(memory_space=pltpu.SMEM),
        pl.BlockSpec(memory_space=pltpu.SMEM),
        pl.BlockSpec((NBLK, 1, BLK), lambda: (0, 0, 0)),
    ],
    out_specs=pl.BlockSpec(memory_space=pltpu.SMEM),
    out_shape=jax.ShapeDtypeStruct((1, 1), jnp.float32),
)


def kernel(logits, text_emb, image_emb, logit_bias):
    bias2d = jnp.reshape(logit_bias, (1, 1)).astype(jnp.float32)
    keys2d, selsim, base_sum = _prep(logits, text_emb, image_emb, bias2d)
    keys = keys2d.reshape(B * B)

    rank0 = jnp.full((1, 1), NEG_RANK, jnp.int32)
    pref0 = jnp.zeros((1, 1), jnp.int32)
    dummy16 = jnp.zeros((L,), jnp.int32)

    hist1 = _sc_pass1(keys, dummy16)
    pref16, rank1, pref1 = _select1(hist1.reshape(NW, L, 4096), rank0, pref0)
    hist2 = _sc_pass2(keys, pref16)
    pref16b, rank2, pref2 = _select2(hist2.reshape(NW, L, 4096), rank1, pref1)
    hist3 = _sc_pass3(keys, pref16b)
    total = _final(hist3.reshape(NW, L, 256), rank2, pref2, base_sum, selsim)
    return jnp.reshape(total, ())


# speedup vs baseline: 128.9762x; 1.0744x over previous
import functools

import jax
import jax.numpy as jnp
from jax import lax
from jax.experimental import pallas as pl
from jax.experimental.pallas import tpu as pltpu
from jax.experimental.pallas import tpu_sc as plsc

B = 4096
D = 64
ALPHA = 0.1
NEG_RANK = (B * B - B - 1) // 2

BLK = 256
NBLK = B // BLK

NW = 32
L = 16
NPER = B * B // NW
CHUNK = 16384
NCH = NPER // CHUNK

INF_KEY = 0xFF800000


def _monotone_key(x):
    u = lax.bitcast_convert_type(x, jnp.uint32)
    return jnp.where((u >> 31) == 0, u | jnp.uint32(0x80000000), ~u)


def _log_sigmoid(z):
    return jnp.minimum(z, 0.0) - jnp.log(1.0 + jnp.exp(-jnp.abs(z)))


def _prep_body(logits_ref, text_ref, img_ref, bias_ref,
               keys_ref, selsim_ref, base_ref):
    i = pl.program_id(0)
    l = logits_ref[...]
    t = text_ref[...]
    im = img_ref[...]
    bias = bias_ref[0, 0]

    raw = lax.dot_general(t, im, (((1,), (1,)), ((), ())),
                          preferred_element_type=jnp.float32)

    rows = i * BLK + lax.broadcasted_iota(jnp.int32, (BLK, B), 0)
    cols = lax.broadcasted_iota(jnp.int32, (BLK, B), 1)
    diag = rows == cols

    lb = l + bias
    z = jnp.where(diag, lb, -lb)

    @pl.when(i == 0)
    def _():
        base_ref[0, 0] = 0.0

    base_ref[0, 0] += jnp.sum(_log_sigmoid(z))

    rm = jnp.max(jnp.where(diag, -jnp.inf, raw), axis=1)
    selsim_ref[0, 0, :] = rm

    ratio = l / (raw + 1e-8)
    key = _monotone_key(ratio)
    key = jnp.where(diag, jnp.uint32(INF_KEY), key)
    keys_ref[...] = key.reshape(BLK * B // 128, 128)


def _prep(logits, text_emb, image_emb, bias2d):
    return pl.pallas_call(
        _prep_body,
        grid=(NBLK,),
        in_specs=[
            pl.BlockSpec((BLK, B), lambda i: (i, 0)),
            pl.BlockSpec((BLK, D), lambda i: (i, 0)),
            pl.BlockSpec((B, D), lambda i: (0, 0)),
            pl.BlockSpec(memory_space=pltpu.SMEM),
        ],
        out_specs=[
            pl.BlockSpec((BLK * B // 128, 128), lambda i: (i, 0)),
            pl.BlockSpec((1, 1, BLK), lambda i: (i, 0, 0)),
            pl.BlockSpec(memory_space=pltpu.SMEM),
        ],
        out_shape=[
            jax.ShapeDtypeStruct((B * B // 128, 128), jnp.uint32),
            jax.ShapeDtypeStruct((NBLK, 1, BLK), jnp.float32),
            jax.ShapeDtypeStruct((1, 1), jnp.float32),
        ],
    )(logits, text_emb, image_emb, bias2d)


def _make_sc_pass(shift, nbins, pshift):
    mesh = plsc.VectorSubcoreMesh(core_axis_name="c", subcore_axis_name="s",
                                  num_cores=2, num_subcores=16)

    def body(keys_hbm, pref_hbm, hist_hbm, buf0, buf1, hist, prefv,
             sem0, sem1):
        cid = lax.axis_index("c")
        sid = lax.axis_index("s")
        wid = sid * 2 + cid
        base = wid * NPER

        zeros = jnp.zeros((L,), jnp.int32)

        @plsc.parallel_loop(0, L * nbins, step=L, unroll=8)
        def _(z_i):
            hist[pl.ds(z_i, L)] = zeros

        lane_base = lax.iota(jnp.int32, L) * nbins
        ones = jnp.ones((L,), jnp.int32)

        if pshift is not None:
            pltpu.sync_copy(pref_hbm, prefv)
            pvu = plsc.bitcast(prefv[...], jnp.uint32)

        def process(buf):
            @plsc.parallel_loop(0, CHUNK, step=L, unroll=8)
            def _(j):
                k = buf[pl.ds(j, L)]
                bin_ = ((k >> shift) & jnp.uint32(nbins - 1)).astype(jnp.int32)
                addr = lane_base + bin_
                if pshift is None:
                    plsc.addupdate_scatter(hist, [addr], ones)
                else:
                    plsc.addupdate_scatter(hist, [addr], ones,
                                           mask=(k >> pshift) == pvu)

        pltpu.make_async_copy(
            keys_hbm.at[pl.ds(base, CHUNK)], buf0, sem0).start()
        pltpu.make_async_copy(
            keys_hbm.at[pl.ds(base + CHUNK, CHUNK)], buf1, sem1).start()

        @pl.loop(0, NCH, step=2)
        def _(c):
            pltpu.make_async_copy(
                keys_hbm.at[pl.ds(base, CHUNK)], buf0, sem0).wait()
            process(buf0)

            @pl.when(c + 2 < NCH)
            def _():
                off = pl.multiple_of(base + (c + 2) * CHUNK, CHUNK)
                pltpu.make_async_copy(
                    keys_hbm.at[pl.ds(off, CHUNK)], buf0, sem0).start()

            pltpu.make_async_copy(
                keys_hbm.at[pl.ds(base, CHUNK)], buf1, sem1).wait()
            process(buf1)

            @pl.when(c + 3 < NCH)
            def _():
                off = pl.multiple_of(base + (c + 3) * CHUNK, CHUNK)
                pltpu.make_async_copy(
                    keys_hbm.at[pl.ds(off, CHUNK)], buf1, sem1).start()

        for lane_i in range(L):
            pltpu.sync_copy(hist.at[pl.ds(lane_i * nbins, nbins)],
                            hist_hbm.at[wid * L + lane_i])

    return pl.kernel(
        body,
        out_type=jax.ShapeDtypeStruct((NW * L, nbins), jnp.int32),
        mesh=mesh,
        scratch_types=[
            pltpu.VMEM((CHUNK,), jnp.uint32),
            pltpu.VMEM((CHUNK,), jnp.uint32),
            pltpu.VMEM((L * nbins,), jnp.int32),
            pltpu.VMEM((L,), jnp.int32),
            pltpu.SemaphoreType.DMA,
            pltpu.SemaphoreType.DMA,
        ],
        compiler_params=pltpu.CompilerParams(needs_layout_passes=False),
    )


_sc_pass1 = _make_sc_pass(20, 4096, None)
_sc_pass2 = _make_sc_pass(8, 4096, 20)
_sc_pass3 = _make_sc_pass(0, 256, 8)


def _cumsum_1d(c, n):
    cum = c
    k = 1
    while k < n:
        shifted = jnp.concatenate(
            [jnp.zeros((1, k), jnp.int32), cum[:, :n - k]], axis=1)
        cum = cum + shifted
        k *= 2
    return cum


def _select_body(nbins, hist_ref, rank_ref, pref_ref,
                 pref16_ref, rank_out, pref_out):
    h = hist_ref[...]
    counts = jnp.sum(h, axis=0)
    cum = _cumsum_1d(counts.reshape(1, nbins), nbins)
    r = rank_ref[0, 0]
    le = cum <= r
    b = jnp.sum(le.astype(jnp.int32))
    prev = jnp.max(jnp.where(le, cum, 0))
    newpref = pref_ref[0, 0] * nbins + b
    rank_out[0, 0] = r - prev
    pref_out[0, 0] = newpref
    pref16_ref[...] = jnp.full((L,), newpref, jnp.int32)


def _make_select(nbins):
    return pl.pallas_call(
        functools.partial(_select_body, nbins),
        in_specs=[
            pl.BlockSpec((NW * L, nbins), lambda: (0, 0)),
            pl.BlockSpec(memory_space=pltpu.SMEM),
            pl.BlockSpec(memory_space=pltpu.SMEM),
        ],
        out_specs=[
            pl.BlockSpec((L,), lambda: (0,)),
            pl.BlockSpec(memory_space=pltpu.SMEM),
            pl.BlockSpec(memory_space=pltpu.SMEM),
        ],
        out_shape=[
            jax.ShapeDtypeStruct((L,), jnp.int32),
            jax.ShapeDtypeStruct((1, 1), jnp.int32),
            jax.ShapeDtypeStruct((1, 1), jnp.int32),
        ],
    )


_select1 = _make_select(4096)
_select2 = _make_select(4096)


def _final_body(hist_ref, rank_ref, pref_ref, base_ref, selsim_ref, out_ref):
    h = hist_ref[...]
    counts = jnp.sum(h, axis=0)
    cum = _cumsum_1d(counts.reshape(1, 256), 256)
    r = rank_ref[0, 0]
    b = jnp.sum((cum <= r).astype(jnp.int32))
    key = pref_ref[0, 0] * 256 + b
    u = jnp.where(key >= 0, ~key, key & jnp.int32(0x7FFFFFFF))
    scale = lax.bitcast_convert_type(u, jnp.float32)

    s = selsim_ref[...]
    select_loss = -jnp.sum(_log_sigmoid(-scale * s)) / B
    base_loss = -base_ref[0, 0] / (B * B)
    out_ref[0, 0] = base_loss + ALPHA * select_loss


_final = pl.pallas_call(
    _final_body,
    in_specs=[
        pl.BlockSpec((NW * L, 256), lambda: (0, 0)),
        pl.BlockSpec(memory_space=pltpu.SMEM),
        pl.BlockSpec(memory_space=pltpu.SMEM),
        pl.BlockSpec(memory_space=pltpu.SMEM),
        pl.BlockSpec((NBLK, 1, BLK), lambda: (0, 0, 0)),
    ],
    out_specs=pl.BlockSpec(memory_space=pltpu.SMEM),
    out_shape=jax.ShapeDtypeStruct((1, 1), jnp.float32),
)


def kernel(logits, text_emb, image_emb, logit_bias):
    bias2d = jnp.reshape(logit_bias, (1, 1)).astype(jnp.float32)
    keys2d, selsim, base_sum = _prep(logits, text_emb, image_emb, bias2d)
    keys = keys2d.reshape(B * B)

    rank0 = jnp.full((1, 1), NEG_RANK, jnp.int32)
    pref0 = jnp.zeros((1, 1), jnp.int32)
    dummy16 = jnp.zeros((L,), jnp.int32)

    hist1 = _sc_pass1(keys, dummy16)
    pref16, rank1, pref1 = _select1(hist1, rank0, pref0)
    hist2 = _sc_pass2(keys, pref16)
    pref16b, rank2, pref2 = _select2(hist2, rank1, pref1)
    hist3 = _sc_pass3(keys, pref16b)
    total = _final(hist3, rank2, pref2, base_sum, selsim)
    return jnp.reshape(total, ())


# speedup vs baseline: 129.2379x; 1.0020x over previous
import functools

import jax
import jax.numpy as jnp
from jax import lax
from jax.experimental import pallas as pl
from jax.experimental.pallas import tpu as pltpu
from jax.experimental.pallas import tpu_sc as plsc

B = 4096
D = 64
ALPHA = 0.1
NEG_RANK = (B * B - B - 1) // 2

BLK = 256
NBLK = B // BLK

NW = 32
L = 16
NPER = B * B // NW
CHUNK = 16384
NCH = NPER // CHUNK

INF_KEY = 0xFF800000


def _monotone_key(x):
    u = lax.bitcast_convert_type(x, jnp.uint32)
    return jnp.where((u >> 31) == 0, u | jnp.uint32(0x80000000), ~u)


def _log_sigmoid(z):
    return jnp.minimum(z, 0.0) - jnp.log(1.0 + jnp.exp(-jnp.abs(z)))


def _prep_body(logits_ref, text_ref, img_ref, bias_ref,
               keys_ref, selsim_ref, base_ref):
    i = pl.program_id(0)
    l = logits_ref[...]
    t = text_ref[...]
    im = img_ref[...]
    bias = bias_ref[0, 0]

    raw = lax.dot_general(t, im, (((1,), (1,)), ((), ())),
                          preferred_element_type=jnp.float32)

    rows = i * BLK + lax.broadcasted_iota(jnp.int32, (BLK, B), 0)
    cols = lax.broadcasted_iota(jnp.int32, (BLK, B), 1)
    diag = rows == cols

    lb = l + bias
    z = jnp.where(diag, lb, -lb)

    @pl.when(i == 0)
    def _():
        base_ref[0, 0] = 0.0

    base_ref[0, 0] += jnp.sum(_log_sigmoid(z))

    rm = jnp.max(jnp.where(diag, -jnp.inf, raw), axis=1)
    selsim_ref[0, 0, :] = rm

    ratio = l / (raw + 1e-8)
    key = _monotone_key(ratio)
    key = jnp.where(diag, jnp.uint32(INF_KEY), key)
    keys_ref[...] = key.reshape(BLK * B // 128, 128)


def _prep(logits, text_emb, image_emb, bias2d):
    return pl.pallas_call(
        _prep_body,
        grid=(NBLK,),
        in_specs=[
            pl.BlockSpec((BLK, B), lambda i: (i, 0)),
            pl.BlockSpec((BLK, D), lambda i: (i, 0)),
            pl.BlockSpec((B, D), lambda i: (0, 0)),
            pl.BlockSpec(memory_space=pltpu.SMEM),
        ],
        out_specs=[
            pl.BlockSpec((BLK * B // 128, 128), lambda i: (i, 0)),
            pl.BlockSpec((1, 1, BLK), lambda i: (i, 0, 0)),
            pl.BlockSpec(memory_space=pltpu.SMEM),
        ],
        out_shape=[
            jax.ShapeDtypeStruct((B * B // 128, 128), jnp.uint32),
            jax.ShapeDtypeStruct((NBLK, 1, BLK), jnp.float32),
            jax.ShapeDtypeStruct((1, 1), jnp.float32),
        ],
    )(logits, text_emb, image_emb, bias2d)


def _make_sc_pass(shift, nbins, pshift):
    mesh = plsc.VectorSubcoreMesh(core_axis_name="c", subcore_axis_name="s",
                                  num_cores=2, num_subcores=16)

    def body(keys_hbm, pref_hbm, hist_hbm, buf0, buf1, hist, prefv,
             sem0, sem1):
        cid = lax.axis_index("c")
        sid = lax.axis_index("s")
        wid = sid * 2 + cid
        base = wid * NPER

        zeros = jnp.zeros((L,), jnp.int32)

        @plsc.parallel_loop(0, L * nbins, step=L, unroll=8)
        def _(z_i):
            hist[pl.ds(z_i, L)] = zeros

        lane_base = lax.iota(jnp.int32, L) * nbins
        ones = jnp.ones((L,), jnp.int32)

        if pshift is not None:
            pltpu.sync_copy(pref_hbm, prefv)
            pvu = plsc.bitcast(prefv[...], jnp.uint32)

        def process(buf):
            @plsc.parallel_loop(0, CHUNK, step=L, unroll=16)
            def _(j):
                k = buf[pl.ds(j, L)]
                bin_ = ((k >> shift) & jnp.uint32(nbins - 1)).astype(jnp.int32)
                addr = lane_base + bin_
                if pshift is None:
                    plsc.addupdate_scatter(hist, [addr], ones)
                else:
                    plsc.addupdate_scatter(hist, [addr], ones,
                                           mask=(k >> pshift) == pvu)

        pltpu.make_async_copy(
            keys_hbm.at[pl.ds(base, CHUNK)], buf0, sem0).start()
        pltpu.make_async_copy(
            keys_hbm.at[pl.ds(base + CHUNK, CHUNK)], buf1, sem1).start()

        @pl.loop(0, NCH, step=2)
        def _(c):
            pltpu.make_async_copy(
                keys_hbm.at[pl.ds(base, CHUNK)], buf0, sem0).wait()
            process(buf0)

            @pl.when(c + 2 < NCH)
            def _():
                off = pl.multiple_of(base + (c + 2) * CHUNK, CHUNK)
                pltpu.make_async_copy(
                    keys_hbm.at[pl.ds(off, CHUNK)], buf0, sem0).start()

            pltpu.make_async_copy(
                keys_hbm.at[pl.ds(base, CHUNK)], buf1, sem1).wait()
            process(buf1)

            @pl.when(c + 3 < NCH)
            def _():
                off = pl.multiple_of(base + (c + 3) * CHUNK, CHUNK)
                pltpu.make_async_copy(
                    keys_hbm.at[pl.ds(off, CHUNK)], buf1, sem1).start()

        for lane_i in range(L):
            pltpu.sync_copy(hist.at[pl.ds(lane_i * nbins, nbins)],
                            hist_hbm.at[wid * L + lane_i])

    return pl.kernel(
        body,
        out_type=jax.ShapeDtypeStruct((NW * L, nbins), jnp.int32),
        mesh=mesh,
        scratch_types=[
            pltpu.VMEM((CHUNK,), jnp.uint32),
            pltpu.VMEM((CHUNK,), jnp.uint32),
            pltpu.VMEM((L * nbins,), jnp.int32),
            pltpu.VMEM((L,), jnp.int32),
            pltpu.SemaphoreType.DMA,
            pltpu.SemaphoreType.DMA,
        ],
        compiler_params=pltpu.CompilerParams(needs_layout_passes=False),
    )


_sc_pass1 = _make_sc_pass(20, 4096, None)
_sc_pass2 = _make_sc_pass(8, 4096, 20)
_sc_pass3 = _make_sc_pass(0, 256, 8)


def _cumsum_1d(c, n):
    cum = c
    k = 1
    while k < n:
        shifted = jnp.concatenate(
            [jnp.zeros((1, k), jnp.int32), cum[:, :n - k]], axis=1)
        cum = cum + shifted
        k *= 2
    return cum


def _select_body(nbins, hist_ref, rank_ref, pref_ref,
                 pref16_ref, rank_out, pref_out):
    h = hist_ref[...]
    counts = jnp.sum(h, axis=0)
    cum = _cumsum_1d(counts.reshape(1, nbins), nbins)
    r = rank_ref[0, 0]
    le = cum <= r
    b = jnp.sum(le.astype(jnp.int32))
    prev = jnp.max(jnp.where(le, cum, 0))
    newpref = pref_ref[0, 0] * nbins + b
    rank_out[0, 0] = r - prev
    pref_out[0, 0] = newpref
    pref16_ref[...] = jnp.full((L,), newpref, jnp.int32)


def _make_select(nbins):
    return pl.pallas_call(
        functools.partial(_select_body, nbins),
        in_specs=[
            pl.BlockSpec((NW * L, nbins), lambda: (0, 0)),
            pl.BlockSpec(memory_space=pltpu.SMEM),
            pl.BlockSpec(memory_space=pltpu.SMEM),
        ],
        out_specs=[
            pl.BlockSpec((L,), lambda: (0,)),
            pl.BlockSpec(memory_space=pltpu.SMEM),
            pl.BlockSpec(memory_space=pltpu.SMEM),
        ],
        out_shape=[
            jax.ShapeDtypeStruct((L,), jnp.int32),
            jax.ShapeDtypeStruct((1, 1), jnp.int32),
            jax.ShapeDtypeStruct((1, 1), jnp.int32),
        ],
    )


_select1 = _make_select(4096)
_select2 = _make_select(4096)


def _final_body(hist_ref, rank_ref, pref_ref, base_ref, selsim_ref, out_ref):
    h = hist_ref[...]
    counts = jnp.sum(h, axis=0)
    cum = _cumsum_1d(counts.reshape(1, 256), 256)
    r = rank_ref[0, 0]
    b = jnp.sum((cum <= r).astype(jnp.int32))
    key = pref_ref[0, 0] * 256 + b
    u = jnp.where(key >= 0, ~key, key & jnp.int32(0x7FFFFFFF))
    scale = lax.bitcast_convert_type(u, jnp.float32)

    s = selsim_ref[...]
    select_loss = -jnp.sum(_log_sigmoid(-scale * s)) / B
    base_loss = -base_ref[0, 0] / (B * B)
    out_ref[0, 0] = base_loss + ALPHA * select_loss


_final = pl.pallas_call(
    _final_body,
    in_specs=[
        pl.BlockSpec((NW * L, 256), lambda: (0, 0)),
        pl.BlockSpec(memory_space=pltpu.SMEM),
        pl.BlockSpec(memory_space=pltpu.SMEM),
        pl.BlockSpec(memory_space=pltpu.SMEM),
        pl.BlockSpec((NBLK, 1, BLK), lambda: (0, 0, 0)),
    ],
    out_specs=pl.BlockSpec(memory_space=pltpu.SMEM),
    out_shape=jax.ShapeDtypeStruct((1, 1), jnp.float32),
)


def kernel(logits, text_emb, image_emb, logit_bias):
    bias2d = jnp.reshape(logit_bias, (1, 1)).astype(jnp.float32)
    keys2d, selsim, base_sum = _prep(logits, text_emb, image_emb, bias2d)
    keys = keys2d.reshape(B * B)

    rank0 = jnp.full((1, 1), NEG_RANK, jnp.int32)
    pref0 = jnp.zeros((1, 1), jnp.int32)
    dummy16 = jnp.zeros((L,), jnp.int32)

    hist1 = _sc_pass1(keys, dummy16)
    pref16, rank1, pref1 = _select1(hist1, rank0, pref0)
    hist2 = _sc_pass2(keys, pref16)
    pref16b, rank2, pref2 = _select2(hist2, rank1, pref1)
    hist3 = _sc_pass3(keys, pref16b)
    total = _final(hist3, rank2, pref2, base_sum, selsim)
    return jnp.reshape(total, ())
